# baseline (speedup 1.0000x reference)
import functools

import jax
import jax.numpy as jnp
from jax import lax
from jax.experimental import pallas as pl
from jax.experimental.pallas import tpu as pltpu
from jax.experimental.pallas import tpu_sc as plsc

D = 64
NW = 32
BT = 32
ST = 25
MPW = ST * BT // NW
NCH = MPW * 8


@jax.jit
def _sc_gather(table, ip_view):
    mesh = plsc.VectorSubcoreMesh(core_axis_name="c", subcore_axis_name="s")

    @functools.partial(
        pl.kernel,
        mesh=mesh,
        out_type=jax.ShapeDtypeStruct((200, 8, BT, 8, 128), jnp.float32),
        scratch_types=[
            pltpu.VMEM((2, 8, 128), jnp.int32),
            pltpu.VMEM((2, 128, D), jnp.float32),
            pltpu.VMEM((2, 8, 8, 128), jnp.float32),
            pltpu.SemaphoreType.DMA,
            pltpu.SemaphoreType.DMA((2,)),
            pltpu.SemaphoreType.DMA((2,)),
        ],
        compiler_params=pltpu.CompilerParams(use_tc_tiling_on_sc=False,
                                             needs_layout_passes=False,
                                             disable_bounds_checks=True),
    )
    def k(table_hbm, ip_hbm, out_hbm, idx_v, rows_v, t_v, isem, gsem, ssem):
        w = lax.axis_index("s") * 2 + lax.axis_index("c")
        iota = lax.iota(jnp.int32, 16)
        lane_grp = [iota + g * 16 for g in range(8)]

        def idx_start(m):
            f = w * MPW + m
            pltpu.async_copy(ip_hbm.at[f // BT, f % BT], idx_v.at[m % 2],
                             isem)

        def idx_wait(m):
            f = w * MPW + m
            pltpu.make_async_copy(ip_hbm.at[f // BT, f % BT],
                                  idx_v.at[m % 2], isem).wait()

        def gather_start(m, si):
            pltpu.async_copy(table_hbm.at[idx_v.at[m % 2, si]],
                             rows_v.at[si % 2], gsem.at[si % 2])

        def gather_wait(m, si):
            pltpu.make_async_copy(table_hbm.at[idx_v.at[m % 2, si]],
                                  rows_v.at[si % 2], gsem.at[si % 2]).wait()

        def out_slice(m, si):
            f = w * MPW + m
            return out_hbm.at[8 * (f // BT) + si, :, f % BT]

        def store_start(m, si):
            pltpu.async_copy(t_v.at[si % 2], out_slice(m, si),
                             ssem.at[si % 2])

        def store_wait(m, si):
            pltpu.make_async_copy(t_v.at[si % 2], out_slice(m, si),
                                  ssem.at[si % 2]).wait()

        def transpose(p):
            rows = rows_v.at[p]
            t = t_v.at[p]

            def body(e4, carry):
                for u in range(4):
                    e = e4 * 4 + u
                    col = jnp.bitwise_and(iota + e, D - 1)
                    te = lax.shift_right_logical(col, 3)
                    ei = jnp.bitwise_and(col, 7)
                    for g in range(8):
                        vals = plsc.load_gather(rows, [lane_grp[g], col])
                        plsc.store_scatter(t, [te, ei, lane_grp[g]], vals)
                return carry

            lax.fori_loop(0, D // 4, body, 0)

        idx_start(0)
        idx_wait(0)
        gather_start(0, 0)

        def step(m, si):
            p = si % 2
            if si >= 2:
                store_wait(m, si - 2)
            else:
                pl.when(m > 0)(lambda: store_wait(m - 1, si + 6))
            if si == 5:
                pl.when(m < MPW - 1)(lambda: idx_start(m + 1))
            if si == 7:

                def _boundary():
                    idx_wait(m + 1)
                    gather_start(m + 1, 0)

                pl.when(m < MPW - 1)(_boundary)
            else:
                gather_start(m, si + 1)
            gather_wait(m, si)
            transpose(p)
            store_start(m, si)

        def body(m, carry):
            for si in range(8):
                step(m, si)
            return carry

        lax.fori_loop(0, MPW, body, 0)

        store_wait(MPW - 1, 6)
        store_wait(MPW - 1, 7)

    return k(table, ip_view)


def kernel(input_pos, pos_enc_table):
    ip = input_pos.astype(jnp.int32)
    ip_view = ip.T.reshape(ST, 8, BT, 128).transpose(0, 2, 1, 3)
    out5 = _sc_gather(pos_enc_table, ip_view)
    return out5.transpose(2, 4, 0, 1, 3).reshape(4096, 200, D)


# speedup vs baseline: 1.6880x; 1.6880x over previous
import functools

import jax
import jax.numpy as jnp
from jax import lax
from jax.experimental import pallas as pl
from jax.experimental.pallas import tpu as pltpu
from jax.experimental.pallas import tpu_sc as plsc

D = 64
NW = 32
BT = 32
ST = 25
MPW = ST * BT // NW
NCH = MPW * 8


@jax.jit
def _sc_gather(table, ip_view):
    mesh = plsc.VectorSubcoreMesh(core_axis_name="c", subcore_axis_name="s")

    @functools.partial(
        pl.kernel,
        mesh=mesh,
        out_type=jax.ShapeDtypeStruct((200, 8, BT, 8, 128), jnp.float32),
        scratch_types=[
            pltpu.VMEM((2, 8, 128), jnp.int32),
            pltpu.VMEM((2, 128, D), jnp.float32),
            pltpu.VMEM((2, 8, 8, 128), jnp.float32),
            pltpu.SemaphoreType.DMA,
            pltpu.SemaphoreType.DMA((2,)),
            pltpu.SemaphoreType.DMA((2,)),
        ],
        compiler_params=pltpu.CompilerParams(use_tc_tiling_on_sc=False,
                                             needs_layout_passes=False,
                                             disable_bounds_checks=True),
    )
    def k(table_hbm, ip_hbm, out_hbm, idx_v, rows_v, t_v, isem, gsem, ssem):
        w = lax.axis_index("s") * 2 + lax.axis_index("c")
        iota = lax.iota(jnp.int32, 16)
        lane_grp = [iota + g * 16 for g in range(8)]

        def idx_start(m):
            f = w * MPW + m
            pltpu.async_copy(ip_hbm.at[f // BT, f % BT], idx_v.at[m % 2],
                             isem)

        def idx_wait(m):
            f = w * MPW + m
            pltpu.make_async_copy(ip_hbm.at[f // BT, f % BT],
                                  idx_v.at[m % 2], isem).wait()

        def gather_start(m, si):
            pltpu.async_copy(table_hbm.at[idx_v.at[m % 2, si]],
                             rows_v.at[si % 2], gsem.at[si % 2])

        def gather_wait(m, si):
            pltpu.make_async_copy(table_hbm.at[idx_v.at[m % 2, si]],
                                  rows_v.at[si % 2], gsem.at[si % 2]).wait()

        def out_slice(m, si):
            f = w * MPW + m
            return out_hbm.at[8 * (f // BT) + si, :, f % BT]

        def store_start(m, si):
            pltpu.async_copy(t_v.at[si % 2], out_slice(m, si),
                             ssem.at[si % 2])

        def store_wait(m, si):
            pltpu.make_async_copy(t_v.at[si % 2], out_slice(m, si),
                                  ssem.at[si % 2]).wait()

        def transpose(p):
            rows = rows_v.at[p]
            t = t_v.at[p]

            def body(e4, carry):
                for u in range(4):
                    e = e4 * 4 + u
                    col = jnp.bitwise_and(iota + e, D - 1)
                    te = lax.shift_right_logical(col, 3)
                    ei = jnp.bitwise_and(col, 7)
                    vals = [plsc.load_gather(rows, [lane_grp[g], col])
                            for g in range(8)]
                    for g in range(8):
                        plsc.store_scatter(t, [te, ei, lane_grp[g]], vals[g])
                return carry

            lax.fori_loop(0, D // 4, body, 0)

        idx_start(0)
        idx_wait(0)
        gather_start(0, 0)

        def step(m, si):
            p = si % 2
            if si >= 2:
                store_wait(m, si - 2)
            else:
                pl.when(m > 0)(lambda: store_wait(m - 1, si + 6))
            if si == 5:
                pl.when(m < MPW - 1)(lambda: idx_start(m + 1))
            if si == 7:

                def _boundary():
                    idx_wait(m + 1)
                    gather_start(m + 1, 0)

                pl.when(m < MPW - 1)(_boundary)
            else:
                gather_start(m, si + 1)
            gather_wait(m, si)
            transpose(p)
            store_start(m, si)

        def body(m, carry):
            for si in range(8):
                step(m, si)
            return carry

        lax.fori_loop(0, MPW, body, 0)

        store_wait(MPW - 1, 6)
        store_wait(MPW - 1, 7)

    return k(table, ip_view)


def kernel(input_pos, pos_enc_table):
    ip = input_pos.astype(jnp.int32)
    ip_view = ip.T.reshape(ST, 8, BT, 128).transpose(0, 2, 1, 3)
    out5 = _sc_gather(pos_enc_table, ip_view)
    return out5.transpose(2, 4, 0, 1, 3).reshape(4096, 200, D)


# speedup vs baseline: 2.1681x; 1.2844x over previous
import functools

import jax
import jax.numpy as jnp
from jax import lax
from jax.experimental import pallas as pl
from jax.experimental.pallas import tpu as pltpu
from jax.experimental.pallas import tpu_sc as plsc

D = 64
NW = 32
BT = 32
ST = 25
MPW = ST * BT // NW
V = 211201
VP = 211216
NFULL = V // 128


@jax.jit
def _sc_table_stage(table_t, last_pad):
    mesh = plsc.VectorSubcoreMesh(core_axis_name="c", subcore_axis_name="s")

    @functools.partial(
        pl.kernel,
        mesh=mesh,
        out_type=jax.ShapeDtypeStruct((VP // 2, 128), jnp.float32),
        scratch_types=[
            pltpu.VMEM((2, D, 128), jnp.float32),
            pltpu.VMEM((2, D, 128), jnp.float32),
            pltpu.SemaphoreType.DMA((2,)),
            pltpu.SemaphoreType.DMA((2,)),
        ],
        compiler_params=pltpu.CompilerParams(use_tc_tiling_on_sc=True,
                                             needs_layout_passes=False,
                                             disable_bounds_checks=True),
    )
    def k(tt_hbm, lp_hbm, out_hbm, buf_v, t_v, gsem, ssem):
        w = lax.axis_index("s") * 2 + lax.axis_index("c")
        iota = lax.iota(jnp.int32, 16)
        lane_grp = [iota + g * 16 for g in range(8)]
        half = [lax.shift_right_logical(lane_grp[g], 1) for g in range(8)]
        hoff = [jnp.bitwise_and(lane_grp[g], 1) * D for g in range(8)]
        nmine = (NFULL - w + NW - 1) // NW

        def slab(j):
            return w + j * NW

        def in_start(j, p):
            pltpu.async_copy(tt_hbm.at[:, pl.ds(slab(j) * 128, 128)],
                             buf_v.at[p], gsem.at[p])

        def in_wait(j, p):
            pltpu.make_async_copy(tt_hbm.at[:, pl.ds(slab(j) * 128, 128)],
                                  buf_v.at[p], gsem.at[p]).wait()

        def out_start(j, p):
            pltpu.async_copy(t_v.at[p], out_hbm.at[pl.ds(slab(j) * 64, 64)],
                             ssem.at[p])

        def out_wait(j, p):
            pltpu.make_async_copy(t_v.at[p],
                                  out_hbm.at[pl.ds(slab(j) * 64, 64)],
                                  ssem.at[p]).wait()

        def transpose(p):
            buf = buf_v.at[p]
            t = t_v.at[p]

            def body(e2, carry):
                for u in range(2):
                    e = e2 * 2 + u
                    ed = jnp.bitwise_and(iota + e, D - 1)
                    vals = [plsc.load_gather(buf, [ed, lane_grp[g]])
                            for g in range(8)]
                    for g in range(8):
                        plsc.store_scatter(t, [half[g], hoff[g] + ed],
                                           vals[g])
                return carry

            lax.fori_loop(0, D // 2, body, 0)

        pl.when(nmine > 0)(lambda: in_start(0, 0))

        def step(j, carry):
            p = j % 2
            pl.when(j >= 2)(lambda: out_wait(j - 2, p))
            pl.when(j + 1 < nmine)(lambda: in_start(j + 1, (j + 1) % 2))
            in_wait(j, p)
            transpose(p)
            out_start(j, p)
            return carry

        lax.fori_loop(0, nmine, step, 0)
        pl.when(nmine >= 2)(lambda: out_wait(nmine - 2, nmine % 2))
        pl.when(nmine >= 1)(lambda: out_wait(nmine - 1, (nmine + 1) % 2))

        def tail():
            pltpu.sync_copy(lp_hbm, buf_v.at[0, pl.ds(0, 8)])
            pltpu.sync_copy(buf_v.at[0, pl.ds(0, 8)],
                            out_hbm.at[pl.ds((V - 1) // 2, 8)])

        pl.when(w == 0)(tail)

    return k(table_t, last_pad)


@jax.jit
def _sc_gather(table216, ip_view):
    mesh = plsc.VectorSubcoreMesh(core_axis_name="c", subcore_axis_name="s")

    @functools.partial(
        pl.kernel,
        mesh=mesh,
        out_type=jax.ShapeDtypeStruct((200, 8, BT, 8, 128), jnp.float32),
        scratch_types=[
            pltpu.VMEM((2, 8, 128), jnp.int32),
            pltpu.VMEM((2, 128, D), jnp.float32),
            pltpu.VMEM((2, 8, 8, 128), jnp.float32),
            pltpu.SemaphoreType.DMA,
            pltpu.SemaphoreType.DMA((2,)),
            pltpu.SemaphoreType.DMA((2,)),
        ],
        compiler_params=pltpu.CompilerParams(use_tc_tiling_on_sc=False,
                                             needs_layout_passes=False,
                                             disable_bounds_checks=True),
    )
    def k(table_hbm, ip_hbm, out_hbm, idx_v, rows_v, t_v, isem, gsem, ssem):
        w = lax.axis_index("s") * 2 + lax.axis_index("c")
        iota = lax.iota(jnp.int32, 16)
        lane_grp = [iota + g * 16 for g in range(8)]

        def idx_start(m):
            f = w * MPW + m
            pltpu.async_copy(ip_hbm.at[f // BT, f % BT], idx_v.at[m % 2],
                             isem)

        def idx_wait(m):
            f = w * MPW + m
            pltpu.make_async_copy(ip_hbm.at[f // BT, f % BT],
                                  idx_v.at[m % 2], isem).wait()

        def gather_start(m, si):
            pltpu.async_copy(table_hbm.at[idx_v.at[m % 2, si]],
                             rows_v.at[si % 2], gsem.at[si % 2])

        def gather_wait(m, si):
            pltpu.make_async_copy(table_hbm.at[idx_v.at[m % 2, si]],
                                  rows_v.at[si % 2], gsem.at[si % 2]).wait()

        def out_slice(m, si):
            f = w * MPW + m
            return out_hbm.at[8 * (f // BT) + si, :, f % BT]

        def store_start(m, si):
            pltpu.async_copy(t_v.at[si % 2], out_slice(m, si),
                             ssem.at[si % 2])

        def store_wait(m, si):
            pltpu.make_async_copy(t_v.at[si % 2], out_slice(m, si),
                                  ssem.at[si % 2]).wait()

        def transpose(p):
            rows = rows_v.at[p]
            t = t_v.at[p]

            def body(e4, carry):
                for u in range(4):
                    e = e4 * 4 + u
                    col = jnp.bitwise_and(iota + e, D - 1)
                    te = lax.shift_right_logical(col, 3)
                    ei = jnp.bitwise_and(col, 7)
                    vals = [plsc.load_gather(rows, [lane_grp[g], col])
                            for g in range(8)]
                    for g in range(8):
                        plsc.store_scatter(t, [te, ei, lane_grp[g]], vals[g])
                return carry

            lax.fori_loop(0, D // 4, body, 0)

        idx_start(0)
        idx_wait(0)
        gather_start(0, 0)

        def step(m, si):
            p = si % 2
            if si >= 2:
                store_wait(m, si - 2)
            else:
                pl.when(m > 0)(lambda: store_wait(m - 1, si + 6))
            if si == 5:
                pl.when(m < MPW - 1)(lambda: idx_start(m + 1))
            if si == 7:

                def _boundary():
                    idx_wait(m + 1)
                    gather_start(m + 1, 0)

                pl.when(m < MPW - 1)(_boundary)
            else:
                gather_start(m, si + 1)
            gather_wait(m, si)
            transpose(p)
            store_start(m, si)

        def body(m, carry):
            for si in range(8):
                step(m, si)
            return carry

        lax.fori_loop(0, MPW, body, 0)

        store_wait(MPW - 1, 6)
        store_wait(MPW - 1, 7)

    return k(table216, ip_view)


def kernel(input_pos, pos_enc_table):
    ip = input_pos.astype(jnp.int32)
    ip_view = ip.T.reshape(ST, 8, BT, 128).transpose(0, 2, 1, 3)
    last_pad = jnp.pad(pos_enc_table[V - 1:], ((0, 7), (0, 64)))
    staged = _sc_table_stage(pos_enc_table.T, last_pad)
    table216 = staged.reshape(VP, D)
    out5 = _sc_gather(table216, ip_view)
    return out5.transpose(2, 4, 0, 1, 3).reshape(4096, 200, D)
